# baseline (speedup 1.0000x reference)
import functools

import jax
import jax.numpy as jnp
from jax import lax
from jax.experimental import pallas as pl
from jax.experimental.pallas import tpu as pltpu
from jax.experimental.pallas import tpu_sc as plsc

NG = 16
DF = 128
NC = 2
NS = 16
NW = NC * NS
CHUNK = 128
BR = 512


NBUF = 2


def _sc_message_pass(g, src2, dst2, npad, nchunk):
    rpt = npad // NS
    nz = rpt // CHUNK
    mesh = plsc.VectorSubcoreMesh(core_axis_name="c", subcore_axis_name="s")

    @functools.partial(
        pl.kernel,
        out_type=jax.ShapeDtypeStruct((NC, npad, DF), jnp.float32),
        mesh=mesh,
        scratch_types=[
            pltpu.VMEM_SHARED((npad, DF), jnp.float32),
            pltpu.VMEM((CHUNK,), jnp.int32),
            pltpu.VMEM((CHUNK,), jnp.int32),
            pltpu.VMEM((CHUNK,), jnp.int32),
            pltpu.VMEM((CHUNK, DF), jnp.float32),
            pltpu.VMEM((CHUNK, DF), jnp.float32),
            pltpu.SemaphoreType.DMA,
            pltpu.SemaphoreType.DMA,
        ],
    )
    def k(g_hbm, src_hbm, dst_hbm, out_hbm, acc, isrc, d0, d1, r0, r1, s0, s1):
        rows = [r0, r1]
        dbuf = [d0, d1]
        sems = [s0, s1]
        c = lax.axis_index("c")
        s = lax.axis_index("s")
        wid = c * NS + s
        rbase = s * rpt

        zb = rows[0]

        def zrow(i, carry):
            for kk in range(DF // 16):
                zb[i, pl.ds(kk * 16, 16)] = jnp.zeros((16,), jnp.float32)
            return carry

        lax.fori_loop(0, CHUNK, zrow, 0)
        for kk in range(nz):
            pltpu.sync_copy(zb, acc.at[pl.ds(rbase + kk * CHUNK, CHUNK)])
        plsc.subcore_barrier()

        ebase = wid * nchunk

        def body(jj, carry):
            j0 = jj * NBUF
            for b in range(NBUF):
                eb = (ebase + j0 + b) * CHUNK
                pltpu.sync_copy(src_hbm.at[pl.ds(eb, CHUNK)], isrc)
                pltpu.sync_copy(dst_hbm.at[pl.ds(eb, CHUNK)], dbuf[b])
                pltpu.async_copy(g_hbm.at[isrc], rows[b], sems[b]).wait()
                pltpu.sync_copy(rows[b], acc.at[dbuf[b]], add=True)
            return carry

        lax.fori_loop(0, nchunk // NBUF, body, 0)
        plsc.subcore_barrier()
        pltpu.sync_copy(acc.at[pl.ds(rbase, rpt)],
                        out_hbm.at[c, pl.ds(rbase, rpt)])

    return k(g, src2, dst2)


def _sc_degree(dst, zeros1, ones_c, npad, epw, nchunk):
    rpt = npad // NS
    mesh = plsc.VectorSubcoreMesh(core_axis_name="c", subcore_axis_name="s")

    @functools.partial(
        pl.kernel,
        out_type=jax.ShapeDtypeStruct((NC, npad), jnp.float32),
        mesh=mesh,
        scratch_types=[
            pltpu.VMEM_SHARED((npad,), jnp.float32),
            pltpu.VMEM((CHUNK,), jnp.int32),
            pltpu.VMEM((CHUNK,), jnp.float32),
        ],
    )
    def k(dst_hbm, z_hbm, ones_hbm, out_hbm, acc, idst, ones_v):
        c = lax.axis_index("c")
        s = lax.axis_index("s")
        wid = c * NS + s
        rbase = s * rpt
        pltpu.sync_copy(z_hbm.at[pl.ds(rbase, rpt)], acc.at[pl.ds(rbase, rpt)])
        pltpu.sync_copy(ones_hbm, ones_v)
        plsc.subcore_barrier()

        ebase = wid * epw

        def body(i, carry):
            b = ebase + i * CHUNK
            pltpu.sync_copy(dst_hbm.at[pl.ds(b, CHUNK)], idst)
            pltpu.sync_copy(ones_v, acc.at[idst], add=True)
            return carry

        lax.fori_loop(0, nchunk, body, 0)
        plsc.subcore_barrier()
        pltpu.sync_copy(acc.at[pl.ds(rbase, rpt)], out_hbm.at[c, pl.ds(rbase, rpt)])

    return k(dst, zeros1, ones_c)


def _prep_body(d_ref, batch_ref, dis_ref, cnt_ref):
    i = pl.program_id(0)
    deg = d_ref[0] + d_ref[1] + 1.0
    dis_ref[...] = lax.rsqrt(deg)
    segs = lax.broadcasted_iota(jnp.int32, (1, NG), 1)
    oh = (batch_ref[...] == segs).astype(jnp.float32)
    ones = jnp.ones((BR, 1), jnp.float32)
    cnt = lax.dot_general(oh, ones, (((0,), (0,)), ((), ())),
                          preferred_element_type=jnp.float32)

    @pl.when(i == 0)
    def _():
        cnt_ref[...] = cnt

    @pl.when(i != 0)
    def _():
        cnt_ref[...] = cnt_ref[...] + cnt


def _tc_prep(dpart, batch2, npad):
    grid = (npad // BR,)
    return pl.pallas_call(
        _prep_body,
        grid=grid,
        in_specs=[
            pl.BlockSpec((NC, BR, 1), lambda i: (0, i, 0)),
            pl.BlockSpec((BR, 1), lambda i: (i, 0)),
        ],
        out_specs=[
            pl.BlockSpec((BR, 1), lambda i: (i, 0)),
            pl.BlockSpec((NG, 1), lambda i: (0, 0)),
        ],
        out_shape=[
            jax.ShapeDtypeStruct((npad, 1), jnp.float32),
            jax.ShapeDtypeStruct((NG, 1), jnp.float32),
        ],
    )(dpart, batch2)


def _head_body(x_ref, dis_ref, w_ref, g_ref):
    g_ref[...] = jnp.dot(x_ref[...], w_ref[...],
                         preferred_element_type=jnp.float32) * dis_ref[...]


def _tc_head(xt, dis, W, npad):
    grid = (npad // BR,)
    return pl.pallas_call(
        _head_body,
        grid=grid,
        in_specs=[
            pl.BlockSpec((BR, DF), lambda i: (i, 0)),
            pl.BlockSpec((BR, 1), lambda i: (i, 0)),
            pl.BlockSpec((DF, DF), lambda i: (0, 0)),
        ],
        out_specs=pl.BlockSpec((BR, DF), lambda i: (i, 0)),
        out_shape=jax.ShapeDtypeStruct((npad, DF), jnp.float32),
    )(xt, dis, W)


def _pools(h, batch_blk, mx_ref, sm_ref, i):
    segs = lax.broadcasted_iota(jnp.int32, (1, NG), 1)
    oh = (batch_blk == segs).astype(jnp.float32)
    smb = lax.dot_general(oh, h, (((0,), (0,)), ((), ())),
                          preferred_element_type=jnp.float32)
    m3 = batch_blk[None] == lax.broadcasted_iota(jnp.int32, (NG, 1, 1), 0)
    mxb = jnp.max(jnp.where(m3, h[None], -jnp.inf), axis=1)

    @pl.when(i == 0)
    def _():
        mx_ref[...] = mxb
        sm_ref[...] = smb

    @pl.when(i != 0)
    def _():
        mx_ref[...] = jnp.maximum(mx_ref[...], mxb)
        sm_ref[...] = sm_ref[...] + smb


def _mid_body(p_ref, gin_ref, dis_ref, b_ref, w_ref, batch_ref,
              g_ref, mx_ref, sm_ref):
    i = pl.program_id(0)
    dis = dis_ref[...]
    h = jnp.maximum((p_ref[0] + p_ref[1] + gin_ref[...]) * dis + b_ref[...],
                    0.0)
    g_ref[...] = jnp.dot(h, w_ref[...],
                         preferred_element_type=jnp.float32) * dis
    _pools(h, batch_ref[...], mx_ref, sm_ref, i)


def _tc_mid(p, gin, dis, b, Wn, batch2, npad):
    grid = (npad // BR,)
    return pl.pallas_call(
        _mid_body,
        grid=grid,
        in_specs=[
            pl.BlockSpec((NC, BR, DF), lambda i: (0, i, 0)),
            pl.BlockSpec((BR, DF), lambda i: (i, 0)),
            pl.BlockSpec((BR, 1), lambda i: (i, 0)),
            pl.BlockSpec((1, DF), lambda i: (0, 0)),
            pl.BlockSpec((DF, DF), lambda i: (0, 0)),
            pl.BlockSpec((BR, 1), lambda i: (i, 0)),
        ],
        out_specs=[
            pl.BlockSpec((BR, DF), lambda i: (i, 0)),
            pl.BlockSpec((NG, DF), lambda i: (0, 0)),
            pl.BlockSpec((NG, DF), lambda i: (0, 0)),
        ],
        out_shape=[
            jax.ShapeDtypeStruct((npad, DF), jnp.float32),
            jax.ShapeDtypeStruct((NG, DF), jnp.float32),
            jax.ShapeDtypeStruct((NG, DF), jnp.float32),
        ],
    )(p, gin, dis, b, Wn, batch2)


def _tail_body(p_ref, gin_ref, dis_ref, b_ref, batch_ref, mx_ref, sm_ref):
    i = pl.program_id(0)
    h = jnp.maximum((p_ref[0] + p_ref[1] + gin_ref[...]) * dis_ref[...]
                    + b_ref[...], 0.0)
    _pools(h, batch_ref[...], mx_ref, sm_ref, i)


def _tc_tail(p, gin, dis, b, batch2, npad):
    grid = (npad // BR,)
    return pl.pallas_call(
        _tail_body,
        grid=grid,
        in_specs=[
            pl.BlockSpec((NC, BR, DF), lambda i: (0, i, 0)),
            pl.BlockSpec((BR, DF), lambda i: (i, 0)),
            pl.BlockSpec((BR, 1), lambda i: (i, 0)),
            pl.BlockSpec((1, DF), lambda i: (0, 0)),
            pl.BlockSpec((BR, 1), lambda i: (i, 0)),
        ],
        out_specs=[
            pl.BlockSpec((NG, DF), lambda i: (0, 0)),
            pl.BlockSpec((NG, DF), lambda i: (0, 0)),
        ],
        out_shape=[
            jax.ShapeDtypeStruct((NG, DF), jnp.float32),
            jax.ShapeDtypeStruct((NG, DF), jnp.float32),
        ],
    )(p, gin, dis, b, batch2)


def kernel(x, edge_index, batch, W1, b1, W2, b2, W3, b3):
    n, df, t_steps = x.shape
    e = edge_index.shape[1]

    npad = ((n + BR - 1) // BR) * BR
    step = NW * CHUNK * NBUF
    ep = ((e + step - 1) // step) * step
    epw = ep // NW
    nchunk = epw // CHUNK

    xpad = jnp.pad(x, ((0, npad - n), (0, 0), (0, 0)))
    batch2 = jnp.pad(batch, (0, npad - n), constant_values=NG).reshape(npad, 1)
    src = jnp.pad(edge_index[0], (0, ep - e))
    dst = jnp.pad(edge_index[1], (0, ep - e), constant_values=npad - 1)
    zeros1 = jnp.zeros((npad,), jnp.float32)
    ones_c = jnp.ones((CHUNK,), jnp.float32)

    dpart = _sc_degree(dst, zeros1, ones_c, npad, epw, nchunk)
    dis, cnt = _tc_prep(dpart.reshape(NC, npad, 1), batch2, npad)

    b1r = b1.reshape(1, DF)
    b2r = b2.reshape(1, DF)
    b3r = b3.reshape(1, DF)

    cntc = jnp.maximum(cnt, 1.0)
    totals = []
    for t in range(t_steps):
        g1 = _tc_head(xpad[:, :, t], dis, W1, npad)
        p1 = _sc_message_pass(g1, src, dst, npad, nchunk)
        g2, mx1, sm1 = _tc_mid(p1, g1, dis, b1r, W2, batch2, npad)
        p2 = _sc_message_pass(g2, src, dst, npad, nchunk)
        g3, mx2, sm2 = _tc_mid(p2, g2, dis, b2r, W3, batch2, npad)
        p3 = _sc_message_pass(g3, src, dst, npad, nchunk)
        mx3, sm3 = _tc_tail(p3, g3, dis, b3r, batch2, npad)
        mx = mx1 + mx2 + mx3
        mean = (sm1 + sm2 + sm3) / cntc
        totals.append(jnp.concatenate([mx, mean], axis=1))
    return jnp.stack(totals, axis=2)


# speedup vs baseline: 1.8069x; 1.8069x over previous
import functools

import jax
import jax.numpy as jnp
from jax import lax
from jax.experimental import pallas as pl
from jax.experimental.pallas import tpu as pltpu
from jax.experimental.pallas import tpu_sc as plsc

NG = 16
DF = 128
NC = 2
NS = 16
NW = NC * NS
CHUNK = 128
BR = 512


NBUF = 1


def _sc_message_pass(g, src2, dst, zeros2, npad, nchunk):
    rpt = npad // NS
    mesh = plsc.VectorSubcoreMesh(core_axis_name="c", subcore_axis_name="s")

    @functools.partial(
        pl.kernel,
        out_type=jax.ShapeDtypeStruct((NC, npad, DF), jnp.float32),
        mesh=mesh,
        scratch_types=[
            pltpu.VMEM_SHARED((npad, DF), jnp.float32),
            pltpu.VMEM((nchunk, CHUNK), jnp.int32),
            pltpu.VMEM((CHUNK,), jnp.int32),
            pltpu.VMEM((CHUNK, DF), jnp.float32),
            pltpu.SemaphoreType.DMA,
        ],
    )
    def k(g_hbm, src_hbm, dst_hbm, z_hbm, out_hbm, acc, isrc, idst, rows, sem):
        c = lax.axis_index("c")
        s = lax.axis_index("s")
        wid = c * NS + s
        rbase = s * rpt

        pltpu.sync_copy(src_hbm.at[wid], isrc)

        @pl.when(c == 0)
        def _():
            pltpu.sync_copy(g_hbm.at[pl.ds(rbase, rpt)], acc.at[pl.ds(rbase, rpt)])

        @pl.when(c != 0)
        def _():
            pltpu.sync_copy(z_hbm.at[pl.ds(rbase, rpt)], acc.at[pl.ds(rbase, rpt)])

        plsc.subcore_barrier()

        ebase = wid * nchunk

        def body(j, carry):
            cp = pltpu.async_copy(g_hbm.at[isrc.at[j]], rows, sem)
            pltpu.sync_copy(dst_hbm.at[pl.ds((ebase + j) * CHUNK, CHUNK)], idst)
            cp.wait()
            pltpu.sync_copy(rows, acc.at[idst], add=True)
            return carry

        lax.fori_loop(0, nchunk, body, 0)
        plsc.subcore_barrier()
        pltpu.sync_copy(acc.at[pl.ds(rbase, rpt)],
                        out_hbm.at[c, pl.ds(rbase, rpt)])

    return k(g, src2, dst, zeros2)


def _sc_degree(dst, zeros1, ones_c, npad, epw, nchunk):
    rpt = npad // NS
    mesh = plsc.VectorSubcoreMesh(core_axis_name="c", subcore_axis_name="s")

    @functools.partial(
        pl.kernel,
        out_type=jax.ShapeDtypeStruct((NC, npad), jnp.float32),
        mesh=mesh,
        scratch_types=[
            pltpu.VMEM_SHARED((npad,), jnp.float32),
            pltpu.VMEM((CHUNK,), jnp.int32),
            pltpu.VMEM((CHUNK,), jnp.float32),
        ],
    )
    def k(dst_hbm, z_hbm, ones_hbm, out_hbm, acc, idst, ones_v):
        c = lax.axis_index("c")
        s = lax.axis_index("s")
        wid = c * NS + s
        rbase = s * rpt
        pltpu.sync_copy(z_hbm.at[pl.ds(rbase, rpt)], acc.at[pl.ds(rbase, rpt)])
        pltpu.sync_copy(ones_hbm, ones_v)
        plsc.subcore_barrier()

        ebase = wid * epw

        def body(i, carry):
            b = ebase + i * CHUNK
            pltpu.sync_copy(dst_hbm.at[pl.ds(b, CHUNK)], idst)
            pltpu.sync_copy(ones_v, acc.at[idst], add=True)
            return carry

        lax.fori_loop(0, nchunk, body, 0)
        plsc.subcore_barrier()
        pltpu.sync_copy(acc.at[pl.ds(rbase, rpt)], out_hbm.at[c, pl.ds(rbase, rpt)])

    return k(dst, zeros1, ones_c)


def _prep_body(d_ref, batch_ref, dis_ref, cnt_ref):
    i = pl.program_id(0)
    deg = d_ref[0] + d_ref[1] + 1.0
    dis_ref[...] = lax.rsqrt(deg)
    segs = lax.broadcasted_iota(jnp.int32, (1, NG), 1)
    oh = (batch_ref[...] == segs).astype(jnp.float32)
    ones = jnp.ones((BR, 1), jnp.float32)
    cnt = lax.dot_general(oh, ones, (((0,), (0,)), ((), ())),
                          preferred_element_type=jnp.float32)

    @pl.when(i == 0)
    def _():
        cnt_ref[...] = cnt

    @pl.when(i != 0)
    def _():
        cnt_ref[...] = cnt_ref[...] + cnt


def _tc_prep(dpart, batch2, npad):
    grid = (npad // BR,)
    return pl.pallas_call(
        _prep_body,
        grid=grid,
        in_specs=[
            pl.BlockSpec((NC, BR, 1), lambda i: (0, i, 0)),
            pl.BlockSpec((BR, 1), lambda i: (i, 0)),
        ],
        out_specs=[
            pl.BlockSpec((BR, 1), lambda i: (i, 0)),
            pl.BlockSpec((NG, 1), lambda i: (0, 0)),
        ],
        out_shape=[
            jax.ShapeDtypeStruct((npad, 1), jnp.float32),
            jax.ShapeDtypeStruct((NG, 1), jnp.float32),
        ],
    )(dpart, batch2)


def _head_body(x_ref, dis_ref, w_ref, g_ref):
    g_ref[...] = jnp.dot(x_ref[...], w_ref[...],
                         preferred_element_type=jnp.float32) * dis_ref[...]


def _tc_head(xt, dis, W, npad):
    grid = (npad // BR,)
    return pl.pallas_call(
        _head_body,
        grid=grid,
        in_specs=[
            pl.BlockSpec((BR, DF), lambda i: (i, 0)),
            pl.BlockSpec((BR, 1), lambda i: (i, 0)),
            pl.BlockSpec((DF, DF), lambda i: (0, 0)),
        ],
        out_specs=pl.BlockSpec((BR, DF), lambda i: (i, 0)),
        out_shape=jax.ShapeDtypeStruct((npad, DF), jnp.float32),
    )(xt, dis, W)


def _pools(h, batch_blk, mx_ref, sm_ref, i):
    segs = lax.broadcasted_iota(jnp.int32, (1, NG), 1)
    oh = (batch_blk == segs).astype(jnp.float32)
    smb = lax.dot_general(oh, h, (((0,), (0,)), ((), ())),
                          preferred_element_type=jnp.float32)
    m3 = batch_blk[None] == lax.broadcasted_iota(jnp.int32, (NG, 1, 1), 0)
    mxb = jnp.max(jnp.where(m3, h[None], -jnp.inf), axis=1)

    @pl.when(i == 0)
    def _():
        mx_ref[...] = mxb
        sm_ref[...] = smb

    @pl.when(i != 0)
    def _():
        mx_ref[...] = jnp.maximum(mx_ref[...], mxb)
        sm_ref[...] = sm_ref[...] + smb


def _mid_body(p_ref, dis_ref, b_ref, w_ref, batch_ref, g_ref, mx_ref, sm_ref):
    i = pl.program_id(0)
    dis = dis_ref[...]
    h = jnp.maximum((p_ref[0] + p_ref[1]) * dis + b_ref[...], 0.0)
    g_ref[...] = jnp.dot(h, w_ref[...],
                         preferred_element_type=jnp.float32) * dis
    _pools(h, batch_ref[...], mx_ref, sm_ref, i)


def _tc_mid(p, dis, b, Wn, batch2, npad):
    grid = (npad // BR,)
    return pl.pallas_call(
        _mid_body,
        grid=grid,
        in_specs=[
            pl.BlockSpec((NC, BR, DF), lambda i: (0, i, 0)),
            pl.BlockSpec((BR, 1), lambda i: (i, 0)),
            pl.BlockSpec((1, DF), lambda i: (0, 0)),
            pl.BlockSpec((DF, DF), lambda i: (0, 0)),
            pl.BlockSpec((BR, 1), lambda i: (i, 0)),
        ],
        out_specs=[
            pl.BlockSpec((BR, DF), lambda i: (i, 0)),
            pl.BlockSpec((NG, DF), lambda i: (0, 0)),
            pl.BlockSpec((NG, DF), lambda i: (0, 0)),
        ],
        out_shape=[
            jax.ShapeDtypeStruct((npad, DF), jnp.float32),
            jax.ShapeDtypeStruct((NG, DF), jnp.float32),
            jax.ShapeDtypeStruct((NG, DF), jnp.float32),
        ],
    )(p, dis, b, Wn, batch2)


def _tail_body(p_ref, dis_ref, b_ref, batch_ref, mx_ref, sm_ref):
    i = pl.program_id(0)
    h = jnp.maximum((p_ref[0] + p_ref[1]) * dis_ref[...] + b_ref[...], 0.0)
    _pools(h, batch_ref[...], mx_ref, sm_ref, i)


def _tc_tail(p, dis, b, batch2, npad):
    grid = (npad // BR,)
    return pl.pallas_call(
        _tail_body,
        grid=grid,
        in_specs=[
            pl.BlockSpec((NC, BR, DF), lambda i: (0, i, 0)),
            pl.BlockSpec((BR, 1), lambda i: (i, 0)),
            pl.BlockSpec((1, DF), lambda i: (0, 0)),
            pl.BlockSpec((BR, 1), lambda i: (i, 0)),
        ],
        out_specs=[
            pl.BlockSpec((NG, DF), lambda i: (0, 0)),
            pl.BlockSpec((NG, DF), lambda i: (0, 0)),
        ],
        out_shape=[
            jax.ShapeDtypeStruct((NG, DF), jnp.float32),
            jax.ShapeDtypeStruct((NG, DF), jnp.float32),
        ],
    )(p, dis, b, batch2)


def kernel(x, edge_index, batch, W1, b1, W2, b2, W3, b3):
    n, df, t_steps = x.shape
    e = edge_index.shape[1]

    npad = ((n + BR - 1) // BR) * BR
    step = NW * CHUNK * NBUF
    ep = ((e + step - 1) // step) * step
    epw = ep // NW
    nchunk = epw // CHUNK

    xpad = jnp.pad(x, ((0, npad - n), (0, 0), (0, 0)))
    batch2 = jnp.pad(batch, (0, npad - n), constant_values=NG).reshape(npad, 1)
    src = jnp.pad(edge_index[0], (0, ep - e))
    dst = jnp.pad(edge_index[1], (0, ep - e), constant_values=npad - 1)
    src2 = src.reshape(NW, nchunk, CHUNK)
    zeros2 = jnp.zeros((npad, DF), jnp.float32)
    zeros1 = jnp.zeros((npad,), jnp.float32)
    ones_c = jnp.ones((CHUNK,), jnp.float32)

    dpart = _sc_degree(dst, zeros1, ones_c, npad, epw, nchunk)
    dis, cnt = _tc_prep(dpart.reshape(NC, npad, 1), batch2, npad)

    b1r = b1.reshape(1, DF)
    b2r = b2.reshape(1, DF)
    b3r = b3.reshape(1, DF)

    cntc = jnp.maximum(cnt, 1.0)
    totals = []
    for t in range(t_steps):
        g1 = _tc_head(xpad[:, :, t], dis, W1, npad)
        p1 = _sc_message_pass(g1, src2, dst, zeros2, npad, nchunk)
        g2, mx1, sm1 = _tc_mid(p1, dis, b1r, W2, batch2, npad)
        p2 = _sc_message_pass(g2, src2, dst, zeros2, npad, nchunk)
        g3, mx2, sm2 = _tc_mid(p2, dis, b2r, W3, batch2, npad)
        p3 = _sc_message_pass(g3, src2, dst, zeros2, npad, nchunk)
        mx3, sm3 = _tc_tail(p3, dis, b3r, batch2, npad)
        mx = mx1 + mx2 + mx3
        mean = (sm1 + sm2 + sm3) / cntc
        totals.append(jnp.concatenate([mx, mean], axis=1))
    return jnp.stack(totals, axis=2)
